# baseline (speedup 1.0000x reference)
import functools

import jax
import jax.numpy as jnp
from jax import lax
from jax.experimental import pallas as pl
from jax.experimental.pallas import tpu as pltpu
from jax.experimental.pallas import tpu_sc as plsc

_CHUNK = 2048
_IPC = _CHUNK // 128
_NTILES = 16
_NPT = 3136
_ZROWS = 112


def _sc_scatter_mean(edata_t, ids_r, n_nodes):
    bsz, d_edge, n_edges = edata_t.shape
    npad = _NTILES * _NPT
    full_chunks = n_edges // _CHUNK
    rem_edges = n_edges - full_chunks * _CHUNK
    rem_rows = rem_edges // 128
    total_chunks = full_chunks + (1 if rem_edges else 0)
    kmax = -(-total_chunks // _NTILES)

    mesh = plsc.VectorSubcoreMesh(core_axis_name="c", subcore_axis_name="s")

    @functools.partial(
        pl.kernel,
        out_type=jax.ShapeDtypeStruct((bsz, npad, d_edge), jnp.float32),
        mesh=mesh,
        scratch_types=[
            pltpu.VMEM((d_edge, _CHUNK), jnp.float32),
            pltpu.VMEM((_CHUNK, d_edge), jnp.float32),
            pltpu.VMEM((_CHUNK,), jnp.int32),
            pltpu.VMEM((128,), jnp.float32),
            pltpu.VMEM((_ZROWS, d_edge), jnp.float32),
            pltpu.VMEM((_NPT,), jnp.float32),
            pltpu.VMEM_SHARED((npad, d_edge), jnp.float32),
            pltpu.VMEM_SHARED((npad,), jnp.float32),
        ],
        compiler_params=pltpu.CompilerParams(
            use_tc_tiling_on_sc=False, needs_layout_passes=False
        ),
    )
    def sc_k(ed_hbm, ids_hbm, agg_hbm, fbuf, ebuf, ibuf, ones, dbuf, cbuf, acc, cnt):
        c = lax.axis_index("c")
        s = lax.axis_index("s")
        base_n = s * _NPT

        zvec = jnp.zeros((d_edge,), jnp.float32)

        def zrow_body(r, _):
            dbuf[r] = zvec
            return 0

        lax.fori_loop(0, _ZROWS, zrow_body, 0)

        z16 = jnp.zeros((16,), jnp.float32)

        def zcnt_body(i, _):
            cbuf[pl.ds(i * 16, 16)] = z16
            return 0

        lax.fori_loop(0, _NPT // 16, zcnt_body, 0)

        o16 = jnp.full((16,), 1.0, jnp.float32)
        for i in range(128 // 16):
            ones[pl.ds(i * 16, 16)] = o16

        for i in range(_NPT // _ZROWS):
            pltpu.sync_copy(dbuf, acc.at[pl.ds(base_n + i * _ZROWS, _ZROWS)])
        pltpu.sync_copy(cbuf, cnt.at[pl.ds(base_n, _NPT)])
        plsc.subcore_barrier()

        fidx = lax.iota(jnp.int32, 16)

        def assemble(n_edge_in_chunk):
            def asm_body(t, _):
                for u in range(16):
                    e = t * 16 + u
                    eidx = jnp.full((16,), 0, jnp.int32) + e
                    ebuf[e] = plsc.load_gather(fbuf, [fidx, eidx])
                return 0

            lax.fori_loop(0, n_edge_in_chunk // 16, asm_body, 0)

        def chunk_body(k, _):
            g = k * _NTILES + s

            @pl.when(g < full_chunks)
            def _full():
                pltpu.sync_copy(ed_hbm.at[c, :, pl.ds(g * _CHUNK, _CHUNK)], fbuf)
                pltpu.sync_copy(ids_hbm.at[c, pl.ds(g * _CHUNK, _CHUNK)], ibuf)
                assemble(_CHUNK)
                for j in range(_IPC):
                    idx = ibuf.at[pl.ds(j * 128, 128)]
                    pltpu.sync_copy(
                        ebuf.at[pl.ds(j * 128, 128)], acc.at[idx], add=True
                    )
                    pltpu.sync_copy(ones, cnt.at[idx], add=True)

            if rem_edges:
                @pl.when(g == full_chunks)
                def _partial():
                    for f in range(d_edge):
                        pltpu.sync_copy(
                            ed_hbm.at[c, f, pl.ds(full_chunks * _CHUNK, rem_edges)],
                            fbuf.at[f, pl.ds(0, rem_edges)],
                        )
                    pltpu.sync_copy(
                        ids_hbm.at[c, pl.ds(full_chunks * _CHUNK, rem_edges)],
                        ibuf.at[pl.ds(0, rem_edges)],
                    )
                    assemble(rem_edges)
                    for j in range(rem_rows):
                        idx = ibuf.at[pl.ds(j * 128, 128)]
                        pltpu.sync_copy(
                            ebuf.at[pl.ds(j * 128, 128)], acc.at[idx], add=True
                        )
                        pltpu.sync_copy(ones, cnt.at[idx], add=True)

            return 0

        lax.fori_loop(0, kmax, chunk_body, 0)
        plsc.subcore_barrier()

        pltpu.sync_copy(cnt.at[pl.ds(base_n, _NPT)], cbuf)

        def recip_body(i, _):
            v = cbuf[pl.ds(i * 16, 16)]
            cbuf[pl.ds(i * 16, 16)] = 1.0 / jnp.maximum(v, 1.0)
            return 0

        lax.fori_loop(0, _NPT // 16, recip_body, 0)

        for i in range(_NPT // _ZROWS):
            pltpu.sync_copy(acc.at[pl.ds(base_n + i * _ZROWS, _ZROWS)], dbuf)

            def div_body(t, _, i=i):
                cvec = cbuf[pl.ds(i * _ZROWS + t * 16, 16)]
                for j in range(16):
                    r = t * 16 + j
                    dbuf[r] = dbuf[r] * jnp.full((d_edge,), cvec[j], jnp.float32)
                return 0

            lax.fori_loop(0, _ZROWS // 16, div_body, 0)
            pltpu.sync_copy(dbuf, agg_hbm.at[c, pl.ds(base_n + i * _ZROWS, _ZROWS)])

    return sc_k(edata_t, ids_r)


def _mlp(agg, vdata, w_e, w_v, bias):
    bsz, npad, d_edge = agg.shape
    n_nodes, d_feat = vdata.shape[1], vdata.shape[2]
    nb = 2048
    grid = (bsz, -(-n_nodes // nb))

    def body(a_ref, v_ref, we_ref, wv_ref, b_ref, o_ref):
        a = a_ref[0]
        v = v_ref[0]
        out = jnp.dot(a, we_ref[...], preferred_element_type=jnp.float32)
        out = out + jnp.dot(v, wv_ref[...], preferred_element_type=jnp.float32)
        out = out + b_ref[...]
        o_ref[0] = jnp.maximum(out, 0.0)

    return pl.pallas_call(
        body,
        grid=grid,
        in_specs=[
            pl.BlockSpec((1, nb, d_edge), lambda b, i: (b, i, 0)),
            pl.BlockSpec((1, nb, d_feat), lambda b, i: (b, i, 0)),
            pl.BlockSpec((d_edge, d_feat), lambda b, i: (0, 0)),
            pl.BlockSpec((d_feat, d_feat), lambda b, i: (0, 0)),
            pl.BlockSpec((1, d_feat), lambda b, i: (0, 0)),
        ],
        out_specs=pl.BlockSpec((1, nb, d_feat), lambda b, i: (b, i, 0)),
        out_shape=jax.ShapeDtypeStruct((bsz, n_nodes, d_feat), jnp.float32),
    )(agg, vdata, w_e, w_v, bias)


def kernel(edata, receiver_ids, vdata, W, b):
    bsz, n_edges, d_edge = edata.shape
    n_nodes = vdata.shape[1]
    ed_t = jnp.transpose(edata, (0, 2, 1))
    agg = _sc_scatter_mean(ed_t, receiver_ids.astype(jnp.int32), n_nodes)
    w_e = W[:d_edge]
    w_v = W[d_edge:]
    return _mlp(agg, vdata, w_e, w_v, b.reshape(1, -1))


# speedup vs baseline: 3.7048x; 3.7048x over previous
import functools

import jax
import jax.numpy as jnp
from jax import lax
from jax.experimental import pallas as pl
from jax.experimental.pallas import tpu as pltpu
from jax.experimental.pallas import tpu_sc as plsc

_CHUNK = 2048
_IPC = _CHUNK // 128
_NTILES = 16
_NPT = 3136
_ZROWS = 112


def _sc_scatter_mean(edata, ids, n_nodes):
    bsz, n_edges, d_edge = edata.shape
    npad = _NTILES * _NPT
    full_chunks = n_edges // _CHUNK
    rem_edges = n_edges - full_chunks * _CHUNK
    rem_rows = rem_edges // 128
    total_chunks = full_chunks + (1 if rem_edges else 0)
    kmax = -(-total_chunks // _NTILES)
    n_pairs = -(-(kmax + 1) // 2)

    mesh = plsc.VectorSubcoreMesh(core_axis_name="c", subcore_axis_name="s")

    @functools.partial(
        pl.kernel,
        out_type=jax.ShapeDtypeStruct((bsz, npad, d_edge), jnp.float32),
        mesh=mesh,
        scratch_types=[
            pltpu.VMEM((_CHUNK, d_edge), jnp.float32),
            pltpu.VMEM((_CHUNK, d_edge), jnp.float32),
            pltpu.VMEM((_CHUNK,), jnp.int32),
            pltpu.VMEM((_CHUNK,), jnp.int32),
            pltpu.VMEM((128,), jnp.float32),
            pltpu.VMEM((_ZROWS, d_edge), jnp.float32),
            pltpu.VMEM((_NPT,), jnp.float32),
            pltpu.VMEM_SHARED((npad, d_edge), jnp.float32),
            pltpu.VMEM_SHARED((npad,), jnp.float32),
            pltpu.SemaphoreType.DMA,
            pltpu.SemaphoreType.DMA,
            pltpu.SemaphoreType.DMA,
        ],
        compiler_params=pltpu.CompilerParams(use_tc_tiling_on_sc=False),
    )
    def sc_k(ed_hbm, ids_hbm, agg_hbm, ebuf0, ebuf1, ibuf0, ibuf1, ones,
             dbuf, cbuf, acc, cnt, sem_l0, sem_l1, sem_s):
        c = lax.axis_index("c")
        s = lax.axis_index("s")
        base_n = s * _NPT

        zvec = jnp.zeros((d_edge,), jnp.float32)

        def zrow_body(r, _):
            dbuf[r] = zvec
            return 0

        lax.fori_loop(0, _ZROWS, zrow_body, 0)

        z16 = jnp.zeros((16,), jnp.float32)

        def zcnt_body(i, _):
            cbuf[pl.ds(i * 16, 16)] = z16
            return 0

        lax.fori_loop(0, _NPT // 16, zcnt_body, 0)

        o16 = jnp.full((16,), 1.0, jnp.float32)
        for i in range(128 // 16):
            ones[pl.ds(i * 16, 16)] = o16

        for i in range(_NPT // _ZROWS):
            pltpu.sync_copy(dbuf, acc.at[pl.ds(base_n + i * _ZROWS, _ZROWS)])
        pltpu.sync_copy(cbuf, cnt.at[pl.ds(base_n, _NPT)])
        plsc.subcore_barrier()

        def load_descs(g, ebuf, ibuf, sem):
            return (
                pltpu.make_async_copy(
                    ed_hbm.at[c, pl.ds(g * _CHUNK, _CHUNK)], ebuf, sem
                ),
                pltpu.make_async_copy(
                    ids_hbm.at[c, pl.ds(g * _CHUNK, _CHUNK)], ibuf, sem
                ),
            )

        def load_descs_rem(ebuf, ibuf, sem):
            return (
                pltpu.make_async_copy(
                    ed_hbm.at[c, pl.ds(full_chunks * _CHUNK, rem_edges)],
                    ebuf.at[pl.ds(0, rem_edges)], sem,
                ),
                pltpu.make_async_copy(
                    ids_hbm.at[c, pl.ds(full_chunks * _CHUNK, rem_edges)],
                    ibuf.at[pl.ds(0, rem_edges)], sem,
                ),
            )

        def issue_loads(g, ebuf, ibuf, sem):
            @pl.when(g < full_chunks)
            def _():
                for d in load_descs(g, ebuf, ibuf, sem):
                    d.start()

            if rem_edges:
                @pl.when(g == full_chunks)
                def _():
                    for d in load_descs_rem(ebuf, ibuf, sem):
                        d.start()

        def wait_loads(g, ebuf, ibuf, sem):
            @pl.when(g < full_chunks)
            def _():
                for d in load_descs(g, ebuf, ibuf, sem):
                    d.wait()

            if rem_edges:
                @pl.when(g == full_chunks)
                def _():
                    for d in load_descs_rem(ebuf, ibuf, sem):
                        d.wait()

        def fire_drain(ebuf, ibuf, nrows):
            hs = []
            for j in range(nrows):
                idx = ibuf.at[pl.ds(j * 128, 128)]
                hs.append(pltpu.async_copy(
                    ebuf.at[pl.ds(j * 128, 128)], acc.at[idx], sem_s, add=True
                ))
                hs.append(pltpu.async_copy(ones, cnt.at[idx], sem_s, add=True))
            for h in hs:
                h.wait()

        def do_scatters(g, ebuf, ibuf):
            @pl.when(g < full_chunks)
            def _():
                fire_drain(ebuf, ibuf, _IPC)

            if rem_edges:
                @pl.when(g == full_chunks)
                def _():
                    fire_drain(ebuf, ibuf, rem_rows)

        def process(g, ebuf, ibuf, sem, nebuf, nibuf, nsem):
            wait_loads(g, ebuf, ibuf, sem)
            issue_loads(g + _NTILES, nebuf, nibuf, nsem)
            do_scatters(g, ebuf, ibuf)

        issue_loads(s, ebuf0, ibuf0, sem_l0)

        def pair_body(kk, _):
            ga = (kk * 2) * _NTILES + s
            process(ga, ebuf0, ibuf0, sem_l0, ebuf1, ibuf1, sem_l1)
            process(ga + _NTILES, ebuf1, ibuf1, sem_l1, ebuf0, ibuf0, sem_l0)
            return 0

        lax.fori_loop(0, n_pairs, pair_body, 0)
        plsc.subcore_barrier()

        pltpu.sync_copy(cnt.at[pl.ds(base_n, _NPT)], cbuf)

        def recip_body(i, _):
            v = cbuf[pl.ds(i * 16, 16)]
            cbuf[pl.ds(i * 16, 16)] = 1.0 / jnp.maximum(v, 1.0)
            return 0

        lax.fori_loop(0, _NPT // 16, recip_body, 0)

        for i in range(_NPT // _ZROWS):
            pltpu.sync_copy(acc.at[pl.ds(base_n + i * _ZROWS, _ZROWS)], dbuf)

            def div_body(t, _, i=i):
                cvec = cbuf[pl.ds(i * _ZROWS + t * 16, 16)]
                for j in range(16):
                    r = t * 16 + j
                    dbuf[r] = dbuf[r] * jnp.full((d_edge,), cvec[j], jnp.float32)
                return 0

            lax.fori_loop(0, _ZROWS // 16, div_body, 0)
            pltpu.sync_copy(dbuf, agg_hbm.at[c, pl.ds(base_n + i * _ZROWS, _ZROWS)])

    return sc_k(edata, ids)


def _mlp(agg, vdata, w_e, w_v, bias):
    bsz, npad, d_edge = agg.shape
    n_nodes, d_feat = vdata.shape[1], vdata.shape[2]
    nb = 2048
    grid = (bsz, -(-n_nodes // nb))

    def body(a_ref, v_ref, we_ref, wv_ref, b_ref, o_ref):
        a = a_ref[0]
        v = v_ref[0]
        out = jnp.dot(a, we_ref[...], preferred_element_type=jnp.float32)
        out = out + jnp.dot(v, wv_ref[...], preferred_element_type=jnp.float32)
        out = out + b_ref[...]
        o_ref[0] = jnp.maximum(out, 0.0)

    return pl.pallas_call(
        body,
        grid=grid,
        in_specs=[
            pl.BlockSpec((1, nb, d_edge), lambda b, i: (b, i, 0)),
            pl.BlockSpec((1, nb, d_feat), lambda b, i: (b, i, 0)),
            pl.BlockSpec((d_edge, d_feat), lambda b, i: (0, 0)),
            pl.BlockSpec((d_feat, d_feat), lambda b, i: (0, 0)),
            pl.BlockSpec((1, d_feat), lambda b, i: (0, 0)),
        ],
        out_specs=pl.BlockSpec((1, nb, d_feat), lambda b, i: (b, i, 0)),
        out_shape=jax.ShapeDtypeStruct((bsz, n_nodes, d_feat), jnp.float32),
    )(agg, vdata, w_e, w_v, bias)


def kernel(edata, receiver_ids, vdata, W, b):
    bsz, n_edges, d_edge = edata.shape
    n_nodes = vdata.shape[1]
    agg = _sc_scatter_mean(edata, receiver_ids.astype(jnp.int32), n_nodes)
    w_e = W[:d_edge]
    w_v = W[d_edge:]
    return _mlp(agg, vdata, w_e, w_v, b.reshape(1, -1))


# speedup vs baseline: 3.7722x; 1.0182x over previous
import functools

import jax
import jax.numpy as jnp
from jax import lax
from jax.experimental import pallas as pl
from jax.experimental.pallas import tpu as pltpu
from jax.experimental.pallas import tpu_sc as plsc

_CHUNK = 2048
_IPC = _CHUNK // 128
_NTILES = 16
_NPT = 3136
_ZROWS = 112


def _sc_scatter_mean(edata, ids, n_nodes):
    bsz, n_edges, d_edge = edata.shape
    npad = _NTILES * _NPT
    full_chunks = n_edges // _CHUNK
    rem_edges = n_edges - full_chunks * _CHUNK
    rem_rows = rem_edges // 128
    total_chunks = full_chunks + (1 if rem_edges else 0)
    kmax = -(-total_chunks // _NTILES)
    n_pairs = -(-(kmax + 1) // 2)

    mesh = plsc.VectorSubcoreMesh(core_axis_name="c", subcore_axis_name="s")

    @functools.partial(
        pl.kernel,
        out_type=jax.ShapeDtypeStruct((bsz, npad, d_edge), jnp.float32),
        mesh=mesh,
        scratch_types=[
            pltpu.VMEM((_CHUNK, d_edge), jnp.float32),
            pltpu.VMEM((_CHUNK, d_edge), jnp.float32),
            pltpu.VMEM((_CHUNK,), jnp.int32),
            pltpu.VMEM((_CHUNK,), jnp.int32),
            pltpu.VMEM((128,), jnp.float32),
            pltpu.VMEM((_ZROWS, d_edge), jnp.float32),
            pltpu.VMEM((_NPT,), jnp.float32),
            pltpu.VMEM_SHARED((npad, d_edge), jnp.float32),
            pltpu.VMEM_SHARED((npad,), jnp.float32),
            pltpu.SemaphoreType.DMA,
            pltpu.SemaphoreType.DMA,
            pltpu.SemaphoreType.DMA,
        ],
        compiler_params=pltpu.CompilerParams(use_tc_tiling_on_sc=False),
    )
    def sc_k(ed_hbm, ids_hbm, agg_hbm, ebuf0, ebuf1, ibuf0, ibuf1, ones,
             dbuf, cbuf, acc, cnt, sem_l0, sem_l1, sem_s):
        c = lax.axis_index("c")
        s = lax.axis_index("s")
        base_n = s * _NPT

        zvec = jnp.zeros((d_edge,), jnp.float32)

        def zrow_body(r, _):
            dbuf[r] = zvec
            return 0

        lax.fori_loop(0, _ZROWS, zrow_body, 0)

        z16 = jnp.zeros((16,), jnp.float32)

        def zcnt_body(i, _):
            cbuf[pl.ds(i * 16, 16)] = z16
            return 0

        lax.fori_loop(0, _NPT // 16, zcnt_body, 0)

        o16 = jnp.full((16,), 1.0, jnp.float32)
        for i in range(128 // 16):
            ones[pl.ds(i * 16, 16)] = o16

        for i in range(_NPT // _ZROWS):
            pltpu.sync_copy(dbuf, acc.at[pl.ds(base_n + i * _ZROWS, _ZROWS)])
        pltpu.sync_copy(cbuf, cnt.at[pl.ds(base_n, _NPT)])
        plsc.subcore_barrier()

        def load_descs(g, ebuf, ibuf, sem):
            return (
                pltpu.make_async_copy(
                    ed_hbm.at[c, pl.ds(g * _CHUNK, _CHUNK)], ebuf, sem
                ),
                pltpu.make_async_copy(
                    ids_hbm.at[c, pl.ds(g * _CHUNK, _CHUNK)], ibuf, sem
                ),
            )

        def load_descs_rem(ebuf, ibuf, sem):
            return (
                pltpu.make_async_copy(
                    ed_hbm.at[c, pl.ds(full_chunks * _CHUNK, rem_edges)],
                    ebuf.at[pl.ds(0, rem_edges)], sem,
                ),
                pltpu.make_async_copy(
                    ids_hbm.at[c, pl.ds(full_chunks * _CHUNK, rem_edges)],
                    ibuf.at[pl.ds(0, rem_edges)], sem,
                ),
            )

        def issue_loads(g, ebuf, ibuf, sem):
            @pl.when(g < full_chunks)
            def _():
                for d in load_descs(g, ebuf, ibuf, sem):
                    d.start()

            if rem_edges:
                @pl.when(g == full_chunks)
                def _():
                    for d in load_descs_rem(ebuf, ibuf, sem):
                        d.start()

        def wait_loads(g, ebuf, ibuf, sem):
            @pl.when(g < full_chunks)
            def _():
                for d in load_descs(g, ebuf, ibuf, sem):
                    d.wait()

            if rem_edges:
                @pl.when(g == full_chunks)
                def _():
                    for d in load_descs_rem(ebuf, ibuf, sem):
                        d.wait()

        def fire_drain(ebuf, ibuf, nrows):
            hs = []
            for j in range(nrows):
                idx = ibuf.at[pl.ds(j * 128, 128)]
                hs.append(pltpu.async_copy(
                    ebuf.at[pl.ds(j * 128, 128)], acc.at[idx], sem_s, add=True
                ))
                hs.append(pltpu.async_copy(ones, cnt.at[idx], sem_s, add=True))
            for h in hs:
                h.wait()

        def do_scatters(g, ebuf, ibuf):
            @pl.when(g < full_chunks)
            def _():
                fire_drain(ebuf, ibuf, _IPC)

            if rem_edges:
                @pl.when(g == full_chunks)
                def _():
                    fire_drain(ebuf, ibuf, rem_rows)

        def process(g, ebuf, ibuf, sem, nebuf, nibuf, nsem):
            wait_loads(g, ebuf, ibuf, sem)
            issue_loads(g + _NTILES, nebuf, nibuf, nsem)
            do_scatters(g, ebuf, ibuf)

        issue_loads(s, ebuf0, ibuf0, sem_l0)

        def pair_body(kk, _):
            ga = (kk * 2) * _NTILES + s
            process(ga, ebuf0, ibuf0, sem_l0, ebuf1, ibuf1, sem_l1)
            process(ga + _NTILES, ebuf1, ibuf1, sem_l1, ebuf0, ibuf0, sem_l0)
            return 0

        lax.fori_loop(0, n_pairs, pair_body, 0)
        plsc.subcore_barrier()

        pltpu.sync_copy(cnt.at[pl.ds(base_n, _NPT)], cbuf)

        def recip_body(i, _):
            v = cbuf[pl.ds(i * 16, 16)]
            cbuf[pl.ds(i * 16, 16)] = 1.0 / jnp.maximum(v, 1.0)
            return 0

        lax.fori_loop(0, _NPT // 16, recip_body, 0)

        for i in range(_NPT // _ZROWS):
            pltpu.sync_copy(acc.at[pl.ds(base_n + i * _ZROWS, _ZROWS)], dbuf)

            def div_body(t, _, i=i):
                cvec = cbuf[pl.ds(i * _ZROWS + t * 16, 16)]
                for j in range(16):
                    r = t * 16 + j
                    dbuf[r] = dbuf[r] * jnp.full((d_edge,), cvec[j], jnp.float32)
                return 0

            lax.fori_loop(0, _ZROWS // 16, div_body, 0)
            pltpu.sync_copy(dbuf, agg_hbm.at[c, pl.ds(base_n + i * _ZROWS, _ZROWS)])

    return sc_k(edata, ids)


def _mlp(agg, vdata, w_e, w_v, bias):
    bsz, npad, d_edge = agg.shape
    n_nodes, d_feat = vdata.shape[1], vdata.shape[2]
    nb = 4096
    grid = (bsz, -(-n_nodes // nb))

    def body(a_ref, v_ref, we_ref, wv_ref, b_ref, o_ref):
        a = a_ref[0]
        v = v_ref[0]
        out = jnp.dot(a, we_ref[...], preferred_element_type=jnp.float32)
        out = out + jnp.dot(v, wv_ref[...], preferred_element_type=jnp.float32)
        out = out + b_ref[...]
        o_ref[0] = jnp.maximum(out, 0.0)

    return pl.pallas_call(
        body,
        grid=grid,
        in_specs=[
            pl.BlockSpec((1, nb, d_edge), lambda b, i: (b, i, 0)),
            pl.BlockSpec((1, nb, d_feat), lambda b, i: (b, i, 0)),
            pl.BlockSpec((d_edge, d_feat), lambda b, i: (0, 0)),
            pl.BlockSpec((d_feat, d_feat), lambda b, i: (0, 0)),
            pl.BlockSpec((1, d_feat), lambda b, i: (0, 0)),
        ],
        out_specs=pl.BlockSpec((1, nb, d_feat), lambda b, i: (b, i, 0)),
        out_shape=jax.ShapeDtypeStruct((bsz, n_nodes, d_feat), jnp.float32),
    )(agg, vdata, w_e, w_v, bias)


def kernel(edata, receiver_ids, vdata, W, b):
    bsz, n_edges, d_edge = edata.shape
    n_nodes = vdata.shape[1]
    agg = _sc_scatter_mean(edata, receiver_ids.astype(jnp.int32), n_nodes)
    w_e = W[:d_edge]
    w_v = W[d_edge:]
    return _mlp(agg, vdata, w_e, w_v, b.reshape(1, -1))


# speedup vs baseline: 3.7897x; 1.0046x over previous
import functools

import jax
import jax.numpy as jnp
from jax import lax
from jax.experimental import pallas as pl
from jax.experimental.pallas import tpu as pltpu
from jax.experimental.pallas import tpu_sc as plsc

_CHUNK = 2048
_IPC = _CHUNK // 128
_NTILES = 16
_NPT = 3136
_ZROWS = 112


def _sc_scatter_mean(edata, ids, n_nodes):
    bsz, n_edges, d_edge = edata.shape
    npad = _NTILES * _NPT
    full_chunks = n_edges // _CHUNK
    rem_edges = n_edges - full_chunks * _CHUNK
    rem_rows = rem_edges // 128
    total_chunks = full_chunks + (1 if rem_edges else 0)
    kmax = -(-total_chunks // _NTILES)
    n_pairs = -(-(kmax + 1) // 2)

    mesh = plsc.VectorSubcoreMesh(core_axis_name="c", subcore_axis_name="s")

    @functools.partial(
        pl.kernel,
        out_type=jax.ShapeDtypeStruct((bsz, npad, d_edge), jnp.float32),
        mesh=mesh,
        scratch_types=[
            pltpu.VMEM((_CHUNK, d_edge), jnp.float32),
            pltpu.VMEM((_CHUNK, d_edge), jnp.float32),
            pltpu.VMEM((_CHUNK,), jnp.int32),
            pltpu.VMEM((_CHUNK,), jnp.int32),
            pltpu.VMEM((128,), jnp.float32),
            pltpu.VMEM((_ZROWS, d_edge), jnp.float32),
            pltpu.VMEM((_NPT,), jnp.float32),
            pltpu.VMEM_SHARED((npad, d_edge), jnp.float32),
            pltpu.VMEM_SHARED((npad,), jnp.float32),
            pltpu.SemaphoreType.DMA,
            pltpu.SemaphoreType.DMA,
            pltpu.SemaphoreType.DMA,
        ],
        compiler_params=pltpu.CompilerParams(use_tc_tiling_on_sc=False),
    )
    def sc_k(ed_hbm, ids_hbm, agg_hbm, ebuf0, ebuf1, ibuf0, ibuf1, ones,
             dbuf, cbuf, acc, cnt, sem_l0, sem_l1, sem_s):
        c = lax.axis_index("c")
        s = lax.axis_index("s")
        base_n = s * _NPT

        zvec = jnp.zeros((d_edge,), jnp.float32)

        def zrow_body(r, _):
            dbuf[r] = zvec
            return 0

        lax.fori_loop(0, _ZROWS, zrow_body, 0)

        z16 = jnp.zeros((16,), jnp.float32)

        def zcnt_body(i, _):
            cbuf[pl.ds(i * 16, 16)] = z16
            return 0

        lax.fori_loop(0, _NPT // 16, zcnt_body, 0)

        o16 = jnp.full((16,), 1.0, jnp.float32)
        for i in range(128 // 16):
            ones[pl.ds(i * 16, 16)] = o16

        for i in range(_NPT // _ZROWS):
            pltpu.sync_copy(dbuf, acc.at[pl.ds(base_n + i * _ZROWS, _ZROWS)])
        pltpu.sync_copy(cbuf, cnt.at[pl.ds(base_n, _NPT)])
        plsc.subcore_barrier()

        def load_descs(g, ebuf, ibuf, sem):
            return (
                pltpu.make_async_copy(
                    ed_hbm.at[c, pl.ds(g * _CHUNK, _CHUNK)], ebuf, sem
                ),
                pltpu.make_async_copy(
                    ids_hbm.at[c, pl.ds(g * _CHUNK, _CHUNK)], ibuf, sem
                ),
            )

        def load_descs_rem(ebuf, ibuf, sem):
            return (
                pltpu.make_async_copy(
                    ed_hbm.at[c, pl.ds(full_chunks * _CHUNK, rem_edges)],
                    ebuf.at[pl.ds(0, rem_edges)], sem,
                ),
                pltpu.make_async_copy(
                    ids_hbm.at[c, pl.ds(full_chunks * _CHUNK, rem_edges)],
                    ibuf.at[pl.ds(0, rem_edges)], sem,
                ),
            )

        def issue_loads(g, ebuf, ibuf, sem):
            @pl.when(g < full_chunks)
            def _():
                for d in load_descs(g, ebuf, ibuf, sem):
                    d.start()

            if rem_edges:
                @pl.when(g == full_chunks)
                def _():
                    for d in load_descs_rem(ebuf, ibuf, sem):
                        d.start()

        def wait_loads(g, ebuf, ibuf, sem):
            @pl.when(g < full_chunks)
            def _():
                for d in load_descs(g, ebuf, ibuf, sem):
                    d.wait()

            if rem_edges:
                @pl.when(g == full_chunks)
                def _():
                    for d in load_descs_rem(ebuf, ibuf, sem):
                        d.wait()

        def fire_drain(ebuf, ibuf, nrows):
            hs = []
            for j in range(nrows):
                idx = ibuf.at[pl.ds(j * 128, 128)]
                hs.append(pltpu.async_copy(
                    ebuf.at[pl.ds(j * 128, 128)], acc.at[idx], sem_s, add=True
                ))
                hs.append(pltpu.async_copy(ones, cnt.at[idx], sem_s, add=True))
            for h in hs:
                h.wait()

        def do_scatters(g, ebuf, ibuf):
            @pl.when(g < full_chunks)
            def _():
                fire_drain(ebuf, ibuf, _IPC)

            if rem_edges:
                @pl.when(g == full_chunks)
                def _():
                    fire_drain(ebuf, ibuf, rem_rows)

        def process(g, ebuf, ibuf, sem, nebuf, nibuf, nsem):
            wait_loads(g, ebuf, ibuf, sem)
            issue_loads(g + _NTILES, nebuf, nibuf, nsem)
            do_scatters(g, ebuf, ibuf)

        issue_loads(s, ebuf0, ibuf0, sem_l0)

        def pair_body(kk, _):
            ga = (kk * 2) * _NTILES + s
            process(ga, ebuf0, ibuf0, sem_l0, ebuf1, ibuf1, sem_l1)
            process(ga + _NTILES, ebuf1, ibuf1, sem_l1, ebuf0, ibuf0, sem_l0)
            return 0

        lax.fori_loop(0, n_pairs, pair_body, 0)
        plsc.subcore_barrier()

        pltpu.sync_copy(cnt.at[pl.ds(base_n, _NPT)], cbuf)

        def recip_body(i, _):
            v = cbuf[pl.ds(i * 16, 16)]
            cbuf[pl.ds(i * 16, 16)] = 1.0 / jnp.maximum(v, 1.0)
            return 0

        lax.fori_loop(0, _NPT // 16, recip_body, 0)

        for i in range(_NPT // _ZROWS):
            pltpu.sync_copy(acc.at[pl.ds(base_n + i * _ZROWS, _ZROWS)], dbuf)

            def div_body(t, _, i=i):
                cvec = cbuf[pl.ds(i * _ZROWS + t * 16, 16)]
                for j in range(16):
                    r = t * 16 + j
                    dbuf[r] = dbuf[r] * jnp.full((d_edge,), cvec[j], jnp.float32)
                return 0

            lax.fori_loop(0, _ZROWS // 16, div_body, 0)
            pltpu.sync_copy(dbuf, agg_hbm.at[c, pl.ds(base_n + i * _ZROWS, _ZROWS)])

    return sc_k(edata, ids)


def _mlp(agg, vdata, w_e, w_v, bias):
    bsz, npad, d_edge = agg.shape
    n_nodes, d_feat = vdata.shape[1], vdata.shape[2]
    nb = 8192
    grid = (bsz, -(-n_nodes // nb))

    def body(a_ref, v_ref, we_ref, wv_ref, b_ref, o_ref):
        a = a_ref[0]
        v = v_ref[0]
        out = jnp.dot(a, we_ref[...], preferred_element_type=jnp.float32)
        out = out + jnp.dot(v, wv_ref[...], preferred_element_type=jnp.float32)
        out = out + b_ref[...]
        o_ref[0] = jnp.maximum(out, 0.0)

    return pl.pallas_call(
        body,
        grid=grid,
        in_specs=[
            pl.BlockSpec((1, nb, d_edge), lambda b, i: (b, i, 0)),
            pl.BlockSpec((1, nb, d_feat), lambda b, i: (b, i, 0)),
            pl.BlockSpec((d_edge, d_feat), lambda b, i: (0, 0)),
            pl.BlockSpec((d_feat, d_feat), lambda b, i: (0, 0)),
            pl.BlockSpec((1, d_feat), lambda b, i: (0, 0)),
        ],
        out_specs=pl.BlockSpec((1, nb, d_feat), lambda b, i: (b, i, 0)),
        out_shape=jax.ShapeDtypeStruct((bsz, n_nodes, d_feat), jnp.float32),
    )(agg, vdata, w_e, w_v, bias)


def kernel(edata, receiver_ids, vdata, W, b):
    bsz, n_edges, d_edge = edata.shape
    n_nodes = vdata.shape[1]
    agg = _sc_scatter_mean(edata, receiver_ids.astype(jnp.int32), n_nodes)
    w_e = W[:d_edge]
    w_v = W[d_edge:]
    return _mlp(agg, vdata, w_e, w_v, b.reshape(1, -1))
